# baseline (speedup 1.0000x reference)
import functools

import jax
import jax.numpy as jnp
from jax import lax
from jax.experimental import pallas as pl
from jax.experimental.pallas import tpu as pltpu
from jax.experimental.pallas import tpu_sc as plsc

N = 10000
E = 320000
H = 128

NC = 2
NS = 16
NW = NC * NS
EPW = E // NW
CH = 125
GC = 20
NG = EPW // (GC * CH)
RPT = 640
N_PAD = NS * RPT
RB = 64
NRB = RPT // RB

_MESH = plsc.VectorSubcoreMesh(core_axis_name="c", subcore_axis_name="s")


def _sc_agg_body(h_hbm, src_hbm, dst_hbm, zslab_hbm, agg_out,
                 src_v, dst_v, rows_a, rows_b, agg_sh,
                 sg_a, sg_b, ss_a, ss_b):
    c = lax.axis_index("c")
    s = lax.axis_index("s")
    wid = s * NC + c
    pltpu.sync_copy(zslab_hbm, agg_sh.at[pl.ds(s * RPT, RPT)])
    plsc.subcore_barrier()

    bufs = (rows_a, rows_b)
    sgs = (sg_a, sg_b)
    sss = (ss_a, ss_b)

    def body(g, carry):
        pltpu.sync_copy(src_hbm.at[wid, g], src_v)
        pltpu.sync_copy(dst_hbm.at[wid, g], dst_v)
        gat = [None, None]
        scat = [None, None]
        gat[0] = pltpu.async_copy(h_hbm.at[src_v.at[0]], bufs[0], sgs[0])
        for j in range(GC):
            p = j % 2
            q = (j + 1) % 2
            if j + 1 < GC:
                if scat[q] is not None:
                    scat[q].wait()
                gat[q] = pltpu.async_copy(h_hbm.at[src_v.at[j + 1]],
                                          bufs[q], sgs[q])
            gat[p].wait()
            scat[p] = pltpu.async_copy(bufs[p], agg_sh.at[dst_v.at[j]],
                                       sss[p], add=True)
        scat[0].wait()
        scat[1].wait()
        return carry

    lax.fori_loop(0, NG, body, 0)
    plsc.subcore_barrier()
    pltpu.sync_copy(agg_sh.at[pl.ds(s * RPT, RPT)],
                    agg_out.at[c, pl.ds(s * RPT, RPT)])


_sc_agg = pl.kernel(
    _sc_agg_body,
    out_type=jax.ShapeDtypeStruct((NC, N_PAD, H), jnp.float32),
    mesh=_MESH,
    scratch_types=[
        pltpu.VMEM((GC, CH), jnp.int32),
        pltpu.VMEM((GC, CH), jnp.int32),
        pltpu.VMEM((CH, H), jnp.float32),
        pltpu.VMEM((CH, H), jnp.float32),
        pltpu.VMEM_SHARED((N_PAD, H), jnp.float32),
        pltpu.SemaphoreType.DMA,
        pltpu.SemaphoreType.DMA,
        pltpu.SemaphoreType.DMA,
        pltpu.SemaphoreType.DMA,
    ],
)


CHD = 128
GCD = 20
NGD = 4
E_PAD = NW * NGD * GCD * CHD


def _sc_deg_body(dst_hbm, ones_hbm, zslab_hbm, deg_out,
                 dst_v, rows_v, deg_sh, sem):
    c = lax.axis_index("c")
    s = lax.axis_index("s")
    wid = s * NC + c
    pltpu.sync_copy(zslab_hbm, deg_sh.at[pl.ds(s * RPT, RPT)])
    pltpu.sync_copy(ones_hbm, rows_v)
    plsc.subcore_barrier()

    def body(g, carry):
        pltpu.sync_copy(dst_hbm.at[wid, g], dst_v)
        descs = []
        for j in range(GCD):
            descs.append(pltpu.async_copy(
                rows_v, deg_sh.at[dst_v.at[j]], sem, add=True))
        for d in descs:
            d.wait()
        return carry

    lax.fori_loop(0, NGD, body, 0)
    plsc.subcore_barrier()
    pltpu.sync_copy(deg_sh.at[pl.ds(s * RPT, RPT)],
                    deg_out.at[c, pl.ds(s * RPT, RPT)])


_sc_deg = pl.kernel(
    _sc_deg_body,
    out_type=jax.ShapeDtypeStruct((NC, N_PAD, H), jnp.float32),
    mesh=_MESH,
    scratch_types=[
        pltpu.VMEM((GCD, CHD), jnp.int32),
        pltpu.VMEM((CHD, H), jnp.float32),
        pltpu.VMEM_SHARED((N_PAD, H), jnp.float32),
        pltpu.SemaphoreType.DMA,
    ],
)


_BN = 1000


def _tc_self_body(h_ref, ws_ref, b_ref, out_ref):
    out_ref[...] = (jnp.dot(h_ref[...], ws_ref[...],
                            preferred_element_type=jnp.float32) + b_ref[...])


def _tc_self(h, ws, b):
    return pl.pallas_call(
        _tc_self_body,
        grid=(N // _BN,),
        in_specs=[
            pl.BlockSpec((_BN, H), lambda i: (i, 0)),
            pl.BlockSpec((H, H), lambda i: (0, 0)),
            pl.BlockSpec((1, H), lambda i: (0, 0)),
        ],
        out_specs=pl.BlockSpec((_BN, H), lambda i: (i, 0)),
        out_shape=jax.ShapeDtypeStruct((N, H), jnp.float32),
    )(h, ws, b.reshape(1, H))


def _tc_combine_body(s_ref, agg_ref, deg_ref, wn_ref, out_ref, *, relu):
    agg = agg_ref[0] + agg_ref[1]
    deg = deg_ref[0, :, 0:1] + deg_ref[1, :, 0:1]
    recip = 1.0 / jnp.maximum(deg, 1.0)
    hn = agg * recip
    out = s_ref[...] + jnp.dot(hn, wn_ref[...],
                               preferred_element_type=jnp.float32)
    out_ref[...] = jnp.maximum(out, 0.0) if relu else out


def _tc_combine(s, agg, deg, wn, relu):
    return pl.pallas_call(
        functools.partial(_tc_combine_body, relu=relu),
        grid=(N // _BN,),
        in_specs=[
            pl.BlockSpec((_BN, H), lambda i: (i, 0)),
            pl.BlockSpec((NC, _BN, H), lambda i: (0, i, 0)),
            pl.BlockSpec((NC, _BN, H), lambda i: (0, i, 0)),
            pl.BlockSpec((H, H), lambda i: (0, 0)),
        ],
        out_specs=pl.BlockSpec((_BN, H), lambda i: (i, 0)),
        out_shape=jax.ShapeDtypeStruct((N, H), jnp.float32),
    )(s, agg, deg, wn)


def kernel(in_feat, edge_index, W0s, W0n, b0, W1s, W1n, b1, W2s, W2n, b2):
    src_flat = edge_index[0].astype(jnp.int32)
    dst_flat = edge_index[1].astype(jnp.int32)
    src = src_flat.reshape(NW, NG, GC, CH)
    dst = dst_flat.reshape(NW, NG, GC, CH)
    dst_pad = jnp.concatenate(
        [dst_flat, jnp.full((E_PAD - E,), N_PAD - 1, jnp.int32)]
    ).reshape(NW, NGD, GCD, CHD)
    zrow = jnp.zeros((RPT, H), jnp.float32)
    ones = jnp.ones((CHD, H), jnp.float32)

    deg = _sc_deg(dst_pad, ones, zrow)
    agg0 = _sc_agg(in_feat, src, dst, zrow)
    s0 = _tc_self(in_feat, W0s, b0)
    h1 = _tc_combine(s0, agg0, deg, W0n, relu=True)
    agg1 = _sc_agg(h1, src, dst, zrow)
    s1 = _tc_self(h1, W1s, b1)
    h2 = _tc_combine(s1, agg1, deg, W1n, relu=True)
    agg2 = _sc_agg(h2, src, dst, zrow)
    s2 = _tc_self(h2, W2s, b2)
    return _tc_combine(s2, agg2, deg, W2n, relu=False)


# speedup vs baseline: 1.0613x; 1.0613x over previous
import functools

import jax
import jax.numpy as jnp
from jax import lax
from jax.experimental import pallas as pl
from jax.experimental.pallas import tpu as pltpu
from jax.experimental.pallas import tpu_sc as plsc

N = 10000
E = 320000
H = 128

NC = 2
NS = 16
NW = NC * NS
EPW = E // NW
CH = 125
GC = 20
NG = EPW // (GC * CH)
RPT = 640
N_PAD = NS * RPT
RB = 64
NRB = RPT // RB

_MESH = plsc.VectorSubcoreMesh(core_axis_name="c", subcore_axis_name="s")


def _sc_agg_body(h_hbm, src_hbm, dst_hbm, zslab_hbm, agg_out,
                 src_a, src_b, dst_a, dst_b, rows_a, rows_b, agg_sh,
                 sz, si_a, si_b, sg_a, sg_b, ss_a, ss_b):
    c = lax.axis_index("c")
    s = lax.axis_index("s")
    wid = s * NC + c
    init = pltpu.async_copy(zslab_hbm, agg_sh.at[pl.ds(s * RPT, RPT)], sz)

    srcs = (src_a, src_b)
    dsts = (dst_a, dst_b)
    bufs = (rows_a, rows_b)
    sis = (si_a, si_b)
    sgs = (sg_a, sg_b)
    sss = (ss_a, ss_b)

    def stage(g):
        p = g % 2
        return (pltpu.async_copy(src_hbm.at[wid, g], srcs[p], sis[p]),
                pltpu.async_copy(dst_hbm.at[wid, g], dsts[p], sis[p]))

    st = [None] * NG
    st[0] = stage(0)
    st[0][0].wait()
    st[0][1].wait()
    init.wait()
    plsc.subcore_barrier()

    gat = [None, None]
    scat = [None, None]
    gat[0] = pltpu.async_copy(h_hbm.at[srcs[0].at[0]], bufs[0], sgs[0])
    for g in range(NG):
        p = g % 2
        for j in range(GC):
            b = (g * GC + j) % 2
            nb = (b + 1) % 2
            if j == 2 and g + 1 < NG:
                st[g + 1] = stage(g + 1)
            nxt = None
            if j + 1 < GC:
                nxt = srcs[p].at[j + 1]
            elif g + 1 < NG:
                st[g + 1][0].wait()
                st[g + 1][1].wait()
                nxt = srcs[(g + 1) % 2].at[0]
            if nxt is not None:
                if scat[nb] is not None:
                    scat[nb].wait()
                gat[nb] = pltpu.async_copy(h_hbm.at[nxt], bufs[nb], sgs[nb])
            gat[b].wait()
            scat[b] = pltpu.async_copy(bufs[b], agg_sh.at[dsts[p].at[j]],
                                       sss[b], add=True)
    scat[0].wait()
    scat[1].wait()
    plsc.subcore_barrier()
    pltpu.sync_copy(agg_sh.at[pl.ds(s * RPT, RPT)],
                    agg_out.at[c, pl.ds(s * RPT, RPT)])


_sc_agg = pl.kernel(
    _sc_agg_body,
    out_type=jax.ShapeDtypeStruct((NC, N_PAD, H), jnp.float32),
    mesh=_MESH,
    scratch_types=[
        pltpu.VMEM((GC, CH), jnp.int32),
        pltpu.VMEM((GC, CH), jnp.int32),
        pltpu.VMEM((GC, CH), jnp.int32),
        pltpu.VMEM((GC, CH), jnp.int32),
        pltpu.VMEM((CH, H), jnp.float32),
        pltpu.VMEM((CH, H), jnp.float32),
        pltpu.VMEM_SHARED((N_PAD, H), jnp.float32),
        pltpu.SemaphoreType.DMA,
        pltpu.SemaphoreType.DMA,
        pltpu.SemaphoreType.DMA,
        pltpu.SemaphoreType.DMA,
        pltpu.SemaphoreType.DMA,
        pltpu.SemaphoreType.DMA,
        pltpu.SemaphoreType.DMA,
    ],
)


CHD = 128
GCD = 20
NGD = 4
E_PAD = NW * NGD * GCD * CHD


def _sc_deg_body(dst_hbm, ones_hbm, zslab_hbm, deg_out,
                 dst_v, rows_v, deg_sh, sem):
    c = lax.axis_index("c")
    s = lax.axis_index("s")
    wid = s * NC + c
    pltpu.sync_copy(zslab_hbm, deg_sh.at[pl.ds(s * RPT, RPT)])
    pltpu.sync_copy(ones_hbm, rows_v)
    plsc.subcore_barrier()

    def body(g, carry):
        pltpu.sync_copy(dst_hbm.at[wid, g], dst_v)
        descs = []
        for j in range(GCD):
            descs.append(pltpu.async_copy(
                rows_v, deg_sh.at[dst_v.at[j]], sem, add=True))
        for d in descs:
            d.wait()
        return carry

    lax.fori_loop(0, NGD, body, 0)
    plsc.subcore_barrier()
    pltpu.sync_copy(deg_sh.at[pl.ds(s * RPT, RPT)],
                    deg_out.at[c, pl.ds(s * RPT, RPT)])


_sc_deg = pl.kernel(
    _sc_deg_body,
    out_type=jax.ShapeDtypeStruct((NC, N_PAD, H), jnp.float32),
    mesh=_MESH,
    scratch_types=[
        pltpu.VMEM((GCD, CHD), jnp.int32),
        pltpu.VMEM((CHD, H), jnp.float32),
        pltpu.VMEM_SHARED((N_PAD, H), jnp.float32),
        pltpu.SemaphoreType.DMA,
    ],
)


_BN = 1000


def _tc_combine_body(h_ref, agg_ref, deg_ref, ws_ref, wn_ref, b_ref, out_ref,
                     *, relu):
    agg = agg_ref[0] + agg_ref[1]
    deg = deg_ref[0, :, 0:1] + deg_ref[1, :, 0:1]
    recip = 1.0 / jnp.maximum(deg, 1.0)
    hn = agg * recip
    out = (jnp.dot(h_ref[...], ws_ref[...], preferred_element_type=jnp.float32)
           + jnp.dot(hn, wn_ref[...], preferred_element_type=jnp.float32)
           + b_ref[...])
    out_ref[...] = jnp.maximum(out, 0.0) if relu else out


def _tc_combine(h, agg, deg, ws, wn, b, relu):
    return pl.pallas_call(
        functools.partial(_tc_combine_body, relu=relu),
        grid=(N // _BN,),
        in_specs=[
            pl.BlockSpec((_BN, H), lambda i: (i, 0)),
            pl.BlockSpec((NC, _BN, H), lambda i: (0, i, 0)),
            pl.BlockSpec((NC, _BN, H), lambda i: (0, i, 0)),
            pl.BlockSpec((H, H), lambda i: (0, 0)),
            pl.BlockSpec((H, H), lambda i: (0, 0)),
            pl.BlockSpec((1, H), lambda i: (0, 0)),
        ],
        out_specs=pl.BlockSpec((_BN, H), lambda i: (i, 0)),
        out_shape=jax.ShapeDtypeStruct((N, H), jnp.float32),
    )(h, agg, deg, ws, wn, b.reshape(1, H))


def kernel(in_feat, edge_index, W0s, W0n, b0, W1s, W1n, b1, W2s, W2n, b2):
    src_flat = edge_index[0].astype(jnp.int32)
    dst_flat = edge_index[1].astype(jnp.int32)
    src = src_flat.reshape(NW, NG, GC, CH)
    dst = dst_flat.reshape(NW, NG, GC, CH)
    dst_pad = jnp.concatenate(
        [dst_flat, jnp.full((E_PAD - E,), N_PAD - 1, jnp.int32)]
    ).reshape(NW, NGD, GCD, CHD)
    zrow = jnp.zeros((RPT, H), jnp.float32)
    ones = jnp.ones((CHD, H), jnp.float32)

    deg = _sc_deg(dst_pad, ones, zrow)
    agg0 = _sc_agg(in_feat, src, dst, zrow)
    h1 = _tc_combine(in_feat, agg0, deg, W0s, W0n, b0, relu=True)
    agg1 = _sc_agg(h1, src, dst, zrow)
    h2 = _tc_combine(h1, agg1, deg, W1s, W1n, b1, relu=True)
    agg2 = _sc_agg(h2, src, dst, zrow)
    return _tc_combine(h2, agg2, deg, W2s, W2n, b2, relu=False)
